# baseline (speedup 1.0000x reference)
import functools

import jax
import jax.numpy as jnp
from jax.experimental import pallas as pl
from jax.experimental.pallas import tpu as pltpu

_NUM_EMB = 1024
_COMMIT = 0.25
_IMGS_PER_STEP = 8


def _vq_one(zt, ew, ewt):
    K = ew.shape[0]
    R = zt.shape[1]

    zwT = jnp.dot(ew, zt, preferred_element_type=jnp.float32)
    z2 = jnp.sum(zt * zt, axis=0, keepdims=True)
    e2 = jnp.sum(ew * ew, axis=1, keepdims=True)
    dT = (z2 - 2.0 * zwT) + e2

    dmin = jnp.min(dT, axis=0, keepdims=True)
    iota0 = jax.lax.broadcasted_iota(jnp.int32, (K, R), 0)
    cand = jnp.where(dT == dmin, iota0, K)
    idx = jnp.min(cand, axis=0, keepdims=True)

    ohT = (iota0 == idx).astype(jnp.float32)
    zqT = jnp.dot(ewt, ohT, preferred_element_type=jnp.float32)

    sq = jnp.sum(dmin)
    cnt = jnp.sum(ohT, axis=1, keepdims=True)
    return zqT, idx, sq, cnt


def _vq_body(z_ref, ew_ref, ewt_ref, zq_ref, idx_ref, loss_ref, perp_ref,
             counts, sq_acc, *, nsteps, total_elems, total_rows):
    i = pl.program_id(0)
    ew = ew_ref[...]
    ewt = ewt_ref[...]

    sq = 0.0
    cnt = None
    for j in range(_IMGS_PER_STEP):
        zt = z_ref[j]
        zqT, idx, sq_j, cnt_j = _vq_one(zt, ew, ewt)
        zq_ref[j] = zt + (zqT - zt)
        idx_ref[j] = idx
        sq = sq + sq_j
        cnt = cnt_j if cnt is None else cnt + cnt_j

    @pl.when(i == 0)
    def _init():
        counts[...] = cnt
        sq_acc[0, 0] = sq

    @pl.when(i != 0)
    def _acc():
        counts[...] = counts[...] + cnt
        sq_acc[0, 0] = sq_acc[0, 0] + sq

    @pl.when(i == nsteps - 1)
    def _final():
        loss = (1.0 + _COMMIT) * sq_acc[0, 0] / total_elems
        loss_ref[...] = jnp.full((1, 1), loss, jnp.float32)
        p = counts[...] * (1.0 / total_rows)
        ent = jnp.sum(p * jnp.log(jnp.maximum(p, 1e-10)), keepdims=True)
        perp_ref[...] = jnp.exp(-ent)


@jax.jit
def _vq(z3, ew, ewt):
    b, c, r = z3.shape
    k = ew.shape[0]
    g = _IMGS_PER_STEP
    nsteps = b // g
    body = functools.partial(
        _vq_body, nsteps=nsteps, total_elems=float(b * c * r),
        total_rows=float(b * r))
    out_shape = (
        jax.ShapeDtypeStruct((b, c, r), jnp.float32),
        jax.ShapeDtypeStruct((b, 1, r), jnp.int32),
        jax.ShapeDtypeStruct((1, 1), jnp.float32),
        jax.ShapeDtypeStruct((1, 1), jnp.float32),
    )
    zq, idx, loss, perp = pl.pallas_call(
        body,
        grid=(nsteps,),
        in_specs=[
            pl.BlockSpec((g, c, r), lambda i: (i, 0, 0)),
            pl.BlockSpec((k, c), lambda i: (0, 0)),
            pl.BlockSpec((c, k), lambda i: (0, 0)),
        ],
        out_specs=(
            pl.BlockSpec((g, c, r), lambda i: (i, 0, 0)),
            pl.BlockSpec((g, 1, r), lambda i: (i, 0, 0)),
            pl.BlockSpec((1, 1), lambda i: (0, 0)),
            pl.BlockSpec((1, 1), lambda i: (0, 0)),
        ),
        out_shape=out_shape,
        scratch_shapes=[
            pltpu.VMEM((k, 1), jnp.float32),
            pltpu.SMEM((1, 1), jnp.float32),
        ],
    )(z3, ew, ewt)
    return zq, idx, loss, perp


def kernel(z_e, emb_w):
    b, c, h, w = z_e.shape
    z3 = z_e.astype(jnp.float32).reshape(b, c, h * w)
    ew = emb_w.astype(jnp.float32)
    zq, idx, loss, perp = _vq(z3, ew, ew.T)
    z_q_st = zq.reshape(b, c, h, w)
    indices = idx.reshape(b, h, w)
    return (z_q_st, loss.reshape(()), perp.reshape(()), indices)


# speedup vs baseline: 1.0268x; 1.0268x over previous
import functools

import jax
import jax.numpy as jnp
from jax.experimental import pallas as pl
from jax.experimental.pallas import tpu as pltpu

_NUM_EMB = 1024
_COMMIT = 0.25
_IMGS_PER_STEP = 4


def _vq_one(zt, ewa, ewt):
    K = ewa.shape[0]
    R = zt.shape[1]

    m = jnp.dot(ewa, zt, preferred_element_type=jnp.float32)
    z2 = jnp.sum(zt * zt, axis=0, keepdims=True)
    e2 = 0.25 * jnp.sum(ewa * ewa, axis=1, keepdims=True)
    dT = (z2 + m) + e2

    dmin = jnp.min(dT, axis=0, keepdims=True)
    iota0 = jax.lax.broadcasted_iota(jnp.int32, (K, R), 0)
    cand = jnp.where(dT == dmin, iota0, K)
    idx = jnp.min(cand, axis=0, keepdims=True)

    ohT = (iota0 == idx).astype(jnp.float32)
    zqT = jnp.dot(ewt, ohT, preferred_element_type=jnp.float32)

    sq = jnp.sum(dmin)
    cnt = jnp.sum(ohT, axis=1, keepdims=True)
    return zqT, idx, sq, cnt


def _vq_body(z_ref, ew_ref, ewt_ref, zq_ref, idx_ref, loss_ref, perp_ref,
             counts, sq_acc, *, nsteps, total_elems, total_rows):
    i = pl.program_id(0)
    ewa = ew_ref[...]
    ewt = ewt_ref[...]

    sq = 0.0
    cnt = None
    for j in range(_IMGS_PER_STEP):
        zt = z_ref[j]
        zqT, idx, sq_j, cnt_j = _vq_one(zt, ewa, ewt)
        zq_ref[j] = zt + (zqT - zt)
        idx_ref[j] = idx
        sq = sq + sq_j
        cnt = cnt_j if cnt is None else cnt + cnt_j

    @pl.when(i == 0)
    def _init():
        counts[...] = cnt
        sq_acc[0, 0] = sq

    @pl.when(i != 0)
    def _acc():
        counts[...] = counts[...] + cnt
        sq_acc[0, 0] = sq_acc[0, 0] + sq

    @pl.when(i == nsteps - 1)
    def _final():
        loss = (1.0 + _COMMIT) * sq_acc[0, 0] / total_elems
        loss_ref[...] = jnp.full((1, 1), loss, jnp.float32)
        p = counts[...] * (1.0 / total_rows)
        ent = jnp.sum(p * jnp.log(jnp.maximum(p, 1e-10)), keepdims=True)
        perp_ref[...] = jnp.exp(-ent)


@jax.jit
def _vq(z3, ewa, ewt):
    b, c, r = z3.shape
    k = ewa.shape[0]
    ca = ewa.shape[1]
    g = _IMGS_PER_STEP
    nsteps = b // g
    body = functools.partial(
        _vq_body, nsteps=nsteps, total_elems=float(b * c * r),
        total_rows=float(b * r))
    out_shape = (
        jax.ShapeDtypeStruct((b, c, r), jnp.float32),
        jax.ShapeDtypeStruct((b, 1, r), jnp.int32),
        jax.ShapeDtypeStruct((1, 1), jnp.float32),
        jax.ShapeDtypeStruct((1, 1), jnp.float32),
    )
    zq, idx, loss, perp = pl.pallas_call(
        body,
        grid=(nsteps,),
        in_specs=[
            pl.BlockSpec((g, c, r), lambda i: (i, 0, 0)),
            pl.BlockSpec((k, ca), lambda i: (0, 0)),
            pl.BlockSpec((c, k), lambda i: (0, 0)),
        ],
        out_specs=(
            pl.BlockSpec((g, c, r), lambda i: (i, 0, 0)),
            pl.BlockSpec((g, 1, r), lambda i: (i, 0, 0)),
            pl.BlockSpec((1, 1), lambda i: (0, 0)),
            pl.BlockSpec((1, 1), lambda i: (0, 0)),
        ),
        out_shape=out_shape,
        scratch_shapes=[
            pltpu.VMEM((k, 1), jnp.float32),
            pltpu.SMEM((1, 1), jnp.float32),
        ],
    )(z3, ewa, ewt)
    return zq, idx, loss, perp


def kernel(z_e, emb_w):
    b, c, h, w = z_e.shape
    z3 = z_e.astype(jnp.float32).reshape(b, c, h * w)
    ew = emb_w.astype(jnp.float32)
    zq, idx, loss, perp = _vq(z3, -2.0 * ew, ew.T)
    z_q_st = zq.reshape(b, c, h, w)
    indices = idx.reshape(b, h, w)
    return (z_q_st, loss.reshape(()), perp.reshape(()), indices)
